# baseline (speedup 1.0000x reference)
import functools

import numpy as np
import jax
import jax.numpy as jnp
from jax import lax
from jax.experimental import pallas as pl
from jax.experimental.pallas import tpu as pltpu
from jax.experimental.pallas import tpu_sc as plsc

_B, _S = 16, 4096
_LANES = 16
_NVEC = _S // _LANES
_RATE = 0.1
_SEED = 42


def _rotl(x, r):
    return ((x << np.uint32(r)) | (x >> np.uint32(32 - r))).astype(np.uint32)


def _threefry2x32(k0, k1, x0, x1):
    x0 = x0.astype(np.uint32).copy()
    x1 = x1.astype(np.uint32).copy()
    ks0, ks1 = np.uint32(k0), np.uint32(k1)
    ks2 = np.uint32(ks0 ^ ks1 ^ np.uint32(0x1BD11BDA))
    rot0, rot1 = (13, 15, 26, 6), (17, 29, 16, 24)
    x0 = (x0 + ks0).astype(np.uint32)
    x1 = (x1 + ks1).astype(np.uint32)
    inj = [(ks1, ks2), (ks2, ks0), (ks0, ks1), (ks1, ks2), (ks2, ks0)]
    for r in range(5):
        for rot in rot0 if r % 2 == 0 else rot1:
            x0 = (x0 + x1).astype(np.uint32)
            x1 = _rotl(x1, rot)
            x1 = (x1 ^ x0).astype(np.uint32)
        a, b = inj[r]
        x0 = (x0 + a).astype(np.uint32)
        x1 = (x1 + b + np.uint32(r + 1)).astype(np.uint32)
    return x0, x1


def _np_bits(key2, n):
    lo = np.arange(n, dtype=np.uint32)
    o0, o1 = _threefry2x32(key2[0], key2[1], np.zeros(n, np.uint32), lo)
    return (o0 ^ o1).astype(np.uint32)


def _np_uniform(key2, shape):
    bits = _np_bits(key2, int(np.prod(shape)))
    fl = ((bits >> np.uint32(9)) | np.uint32(0x3F800000)).view(np.float32)
    return (fl - np.float32(1.0)).reshape(shape)


def _keep_mask():
    o0, o1 = _threefry2x32(0, _SEED, np.zeros(2, np.uint32),
                           np.arange(2, dtype=np.uint32))
    k1, k2 = (o0[0], o1[0]), (o0[1], o1[1])
    u = _np_uniform(k1, (_B, _S))
    num = (u < np.float32(_RATE)).sum(axis=1).astype(np.int64)
    skeys = _np_uniform(k2, (_B, _S))
    perm = np.argsort(skeys, axis=1, kind="stable")
    ranks = np.argsort(perm, axis=1, kind="stable")
    keep = ranks >= num[:, None]
    return keep.astype(np.int32)


_KEEP = _keep_mask()
_TAIL0 = (int(_KEEP.sum(axis=1).min()) // _LANES) * _LANES

_need = int(np.max(np.argmax(np.cumsum(_KEEP, axis=1) >= _S // 2, axis=1))) + 1
_NB = ((_need + _LANES - 1) // _LANES + 15) // 16 * 16
_B1 = _S // _LANES - _NB
assert _B1 * _LANES <= _S // 2 and _NB * _LANES <= _S
assert _TAIL0 >= _S // 2

_mesh = plsc.VectorSubcoreMesh(core_axis_name="c", subcore_axis_name="s")


@functools.partial(
    pl.kernel,
    mesh=_mesh,
    compiler_params=pltpu.CompilerParams(
        needs_layout_passes=False,
        disable_bounds_checks=True,
        disable_semaphore_checks=True,
        skip_device_barrier=True,
    ),
    out_type=[
        jax.ShapeDtypeStruct((_B, _S), jnp.int32),
        jax.ShapeDtypeStruct((_B, _LANES), jnp.int32),
    ],
    scratch_types=[
        pltpu.VMEM((_S,), jnp.int32),
        pltpu.VMEM((_S,), jnp.int32),
        pltpu.VMEM((_S,), jnp.int32),
        pltpu.VMEM((_LANES,), jnp.int32),
        pltpu.VMEM((_LANES,), jnp.int32),
        pltpu.SemaphoreType.DMA,
    ],
)
def _compact(tok_hbm, msk_hbm, zero_hbm, c1_hbm, out_hbm, len_hbm,
             tok_v, msk_v, out_v, len_v, c1_v, sem):
    wid = lax.axis_index("s") * 2 + lax.axis_index("c")
    b = lax.shift_right_logical(wid, 1)
    h = lax.rem(wid, 2)

    def run(lo_blk, hi_blk, carry0):
        @plsc.parallel_loop(lo_blk, hi_blk, step=1, unroll=16, carry=carry0)
        def total(i, carry):
            sl = pl.ds(i * _LANES, _LANES)
            tok = tok_v[sl]
            m = msk_v[sl]
            mb = m != 0
            incl = plsc.cumsum(m)
            pos = carry + (incl - m)
            plsc.store_scatter(out_v, [pos], tok, mask=mb)
            return carry + plsc.all_reduce_population_count(mb)
        return total

    @pl.when(h == 0)
    def _():
        n = _NB * _LANES
        d1 = pltpu.async_copy(tok_hbm.at[b, pl.ds(0, n)],
                              tok_v.at[pl.ds(0, n)], sem)
        d2 = pltpu.async_copy(msk_hbm.at[b, pl.ds(0, n)],
                              msk_v.at[pl.ds(0, n)], sem)
        d1.wait()
        d2.wait()
        run(0, _NB, jnp.zeros((_LANES,), jnp.int32))
        pltpu.sync_copy(out_v.at[pl.ds(0, _S // 2)],
                        out_hbm.at[b, pl.ds(0, _S // 2)])

    @pl.when(h == 1)
    def _():
        base = _B1 * _LANES
        d1 = pltpu.async_copy(tok_hbm.at[b, pl.ds(base, _S - base)],
                              tok_v.at[pl.ds(base, _S - base)], sem)
        d2 = pltpu.async_copy(msk_hbm.at[b, pl.ds(base, _S - base)],
                              msk_v.at[pl.ds(base, _S - base)], sem)
        d3 = pltpu.async_copy(zero_hbm, out_v.at[pl.ds(_TAIL0, _S - _TAIL0)],
                              sem)
        d4 = pltpu.async_copy(c1_hbm.at[b], c1_v, sem)
        d1.wait()
        d2.wait()
        d3.wait()
        d4.wait()
        total = run(_B1, _NVEC, c1_v[...])
        len_v[...] = total
        pltpu.sync_copy(out_v.at[pl.ds(_S // 2, _S // 2)],
                        out_hbm.at[b, pl.ds(_S // 2, _S // 2)])
        pltpu.sync_copy(len_v, len_hbm.at[b])


_ZROW = np.zeros((_S - _TAIL0,), np.int32)
_C1ROW = np.repeat(
    _KEEP[:, : _B1 * _LANES].sum(axis=1).astype(np.int32)[:, None],
    _LANES, axis=1)


def kernel(inputs):
    out, lens = _compact(inputs, jnp.asarray(_KEEP), jnp.asarray(_ZROW),
                         jnp.asarray(_C1ROW))
    return out, lens[:, 0]


# speedup vs baseline: 1.0394x; 1.0394x over previous
import functools

import numpy as np
import jax
import jax.numpy as jnp
from jax import lax
from jax.experimental import pallas as pl
from jax.experimental.pallas import tpu as pltpu
from jax.experimental.pallas import tpu_sc as plsc

_B, _S = 16, 4096
_LANES = 16
_NVEC = _S // _LANES
_RATE = 0.1
_SEED = 42


def _rotl(x, r):
    return ((x << np.uint32(r)) | (x >> np.uint32(32 - r))).astype(np.uint32)


def _threefry2x32(k0, k1, x0, x1):
    x0 = x0.astype(np.uint32).copy()
    x1 = x1.astype(np.uint32).copy()
    ks0, ks1 = np.uint32(k0), np.uint32(k1)
    ks2 = np.uint32(ks0 ^ ks1 ^ np.uint32(0x1BD11BDA))
    rot0, rot1 = (13, 15, 26, 6), (17, 29, 16, 24)
    x0 = (x0 + ks0).astype(np.uint32)
    x1 = (x1 + ks1).astype(np.uint32)
    inj = [(ks1, ks2), (ks2, ks0), (ks0, ks1), (ks1, ks2), (ks2, ks0)]
    for r in range(5):
        for rot in rot0 if r % 2 == 0 else rot1:
            x0 = (x0 + x1).astype(np.uint32)
            x1 = _rotl(x1, rot)
            x1 = (x1 ^ x0).astype(np.uint32)
        a, b = inj[r]
        x0 = (x0 + a).astype(np.uint32)
        x1 = (x1 + b + np.uint32(r + 1)).astype(np.uint32)
    return x0, x1


def _np_bits(key2, n):
    lo = np.arange(n, dtype=np.uint32)
    o0, o1 = _threefry2x32(key2[0], key2[1], np.zeros(n, np.uint32), lo)
    return (o0 ^ o1).astype(np.uint32)


def _np_uniform(key2, shape):
    bits = _np_bits(key2, int(np.prod(shape)))
    fl = ((bits >> np.uint32(9)) | np.uint32(0x3F800000)).view(np.float32)
    return (fl - np.float32(1.0)).reshape(shape)


def _keep_mask():
    o0, o1 = _threefry2x32(0, _SEED, np.zeros(2, np.uint32),
                           np.arange(2, dtype=np.uint32))
    k1, k2 = (o0[0], o1[0]), (o0[1], o1[1])
    u = _np_uniform(k1, (_B, _S))
    num = (u < np.float32(_RATE)).sum(axis=1).astype(np.int64)
    skeys = _np_uniform(k2, (_B, _S))
    perm = np.argsort(skeys, axis=1, kind="stable")
    ranks = np.argsort(perm, axis=1, kind="stable")
    keep = ranks >= num[:, None]
    return keep.astype(np.int32)


_KEEP = _keep_mask()
_TAIL0 = (int(_KEEP.sum(axis=1).min()) // _LANES) * _LANES

_mesh = plsc.VectorSubcoreMesh(core_axis_name="c", subcore_axis_name="s")


@functools.partial(
    pl.kernel,
    mesh=_mesh,
    compiler_params=pltpu.CompilerParams(
        needs_layout_passes=False,
        disable_bounds_checks=True,
        disable_semaphore_checks=True,
        skip_device_barrier=True,
    ),
    out_type=jax.ShapeDtypeStruct((_B, _S), jnp.int32),
    scratch_types=[
        pltpu.VMEM((_S,), jnp.int32),
        pltpu.VMEM((_S,), jnp.int32),
        pltpu.VMEM((_S,), jnp.int32),
        pltpu.SemaphoreType.DMA,
    ],
)
def _compact(tok_hbm, msk_hbm, zero_hbm, out_hbm,
             tok_v, msk_v, out_v, sem):
    wid = lax.axis_index("s") * 2 + lax.axis_index("c")

    @pl.when(wid < _B)
    def _():
        c1 = pltpu.async_copy(tok_hbm.at[wid], tok_v, sem)
        c2 = pltpu.async_copy(msk_hbm.at[wid], msk_v, sem)
        c3 = pltpu.async_copy(zero_hbm, out_v.at[pl.ds(_TAIL0, _S - _TAIL0)],
                              sem)
        c1.wait()
        c2.wait()
        c3.wait()

        @plsc.parallel_loop(0, _NVEC, step=1, unroll=16,
                            carry=jnp.zeros((_LANES,), jnp.int32))
        def total(i, carry):
            sl = pl.ds(i * _LANES, _LANES)
            tok = tok_v[sl]
            m = msk_v[sl]
            mb = m != 0
            incl = plsc.cumsum(m)
            pos = carry + (incl - m)
            plsc.store_scatter(out_v, [pos], tok, mask=mb)
            return carry + plsc.all_reduce_population_count(mb)
        del total
        pltpu.sync_copy(out_v, out_hbm.at[wid])


_ZROW = np.zeros((_S - _TAIL0,), np.int32)
_LENS = _KEEP.sum(axis=1).astype(np.int32)


def kernel(inputs):
    out = _compact(inputs, jnp.asarray(_KEEP), jnp.asarray(_ZROW))
    return out, jnp.asarray(_LENS)
